# Initial kernel scaffold; baseline (speedup 1.0000x reference)
import functools

import jax
import jax.numpy as jnp
from jax import lax
from jax.experimental import pallas as pl
from jax.experimental.pallas import tpu as pltpu
from jax.experimental.pallas import tpu_sc as plsc

NS = 16
LANES = 16
CHUNK = 128


def _t1_body(x_ref, w0_ref, b0_ref, wa_ref, wb_ref, o0_ref, oa_ref, ob_ref):
    x = x_ref[...]
    o0_ref[...] = jnp.dot(x, w0_ref[...], preferred_element_type=jnp.float32) + b0_ref[...]
    oa_ref[...] = jnp.dot(x, wa_ref[...], preferred_element_type=jnp.float32)
    ob_ref[...] = jnp.dot(x, wb_ref[...], preferred_element_type=jnp.float32)


def _t1(x, W0, b0, Wa, Wb):
    n, d = x.shape
    h = W0.shape[1]
    blk = 2000
    grid = n // blk
    return pl.pallas_call(
        _t1_body,
        grid=(grid,),
        in_specs=[
            pl.BlockSpec((blk, d), lambda i: (i, 0)),
            pl.BlockSpec((d, h), lambda i: (0, 0)),
            pl.BlockSpec((1, h), lambda i: (0, 0)),
            pl.BlockSpec((d, h), lambda i: (0, 0)),
            pl.BlockSpec((d, h), lambda i: (0, 0)),
        ],
        out_specs=[
            pl.BlockSpec((blk, h), lambda i: (i, 0)),
            pl.BlockSpec((blk, h), lambda i: (i, 0)),
            pl.BlockSpec((blk, h), lambda i: (i, 0)),
        ],
        out_shape=[
            jax.ShapeDtypeStruct((n, h), jnp.float32),
            jax.ShapeDtypeStruct((n, h), jnp.float32),
            jax.ShapeDtypeStruct((n, h), jnp.float32),
        ],
    )(x, W0, b0.reshape(1, h), Wa, Wb)


def _bn_mm3_body(x0_ref, pa_ref, pb_ref, bsum_ref, g_ref, bb_ref,
                 w0_ref, b0_ref, wa_ref, wb_ref, scale_ref,
                 o0_ref, oa_ref, ob_ref):
    left = x0_ref[...]
    right = scale_ref[0] * (pa_ref[...] + pb_ref[...] + bsum_ref[...])
    u = jnp.concatenate([left, right], axis=1)
    nrows = u.shape[0]
    mu = jnp.sum(u, axis=0, keepdims=True) / nrows
    xc = u - mu
    var = jnp.sum(xc * xc, axis=0, keepdims=True) / nrows
    z = xc * lax.rsqrt(var + 1e-5) * g_ref[...] + bb_ref[...]
    o0_ref[...] = jnp.dot(z, w0_ref[...], preferred_element_type=jnp.float32) + b0_ref[...]
    oa_ref[...] = jnp.dot(z, wa_ref[...], preferred_element_type=jnp.float32)
    ob_ref[...] = jnp.dot(z, wb_ref[...], preferred_element_type=jnp.float32)


def _t2(x0, pa, pb, bsum, scale, g, bb, W0, b0, Wa, Wb):
    n, h = x0.shape
    c = W0.shape[1]
    return pl.pallas_call(
        _bn_mm3_body,
        out_shape=[
            jax.ShapeDtypeStruct((n, c), jnp.float32),
            jax.ShapeDtypeStruct((n, c), jnp.float32),
            jax.ShapeDtypeStruct((n, c), jnp.float32),
        ],
    )(x0, pa, pb, bsum.reshape(1, h), g.reshape(1, 2 * h), bb.reshape(1, 2 * h),
      W0, b0.reshape(1, c), Wa, Wb, jnp.asarray([scale], jnp.float32))


def _bn_final_body(x0_ref, pa_ref, pb_ref, bsum_ref, g_ref, bb_ref,
                   wt_ref, bout_ref, scale_ref, o_ref):
    left = x0_ref[...]
    right = scale_ref[0] * (pa_ref[...] + pb_ref[...] + bsum_ref[...])
    u = jnp.concatenate([left, right], axis=1)
    nrows = u.shape[0]
    mu = jnp.sum(u, axis=0, keepdims=True) / nrows
    xc = u - mu
    var = jnp.sum(xc * xc, axis=0, keepdims=True) / nrows
    z = xc * lax.rsqrt(var + 1e-5) * g_ref[...] + bb_ref[...]
    o_ref[...] = jnp.dot(z, wt_ref[...], preferred_element_type=jnp.float32) + bout_ref[...]


def _t3(x0, pa, pb, bsum, scale, g, bb, Wt, bout):
    n, c = x0.shape
    cout = Wt.shape[1]
    return pl.pallas_call(
        _bn_final_body,
        out_shape=jax.ShapeDtypeStruct((n, cout), jnp.float32),
    )(x0, pa, pb, bsum.reshape(1, c), g.reshape(1, 2 * c), bb.reshape(1, 2 * c),
      Wt, bout.reshape(1, cout), jnp.asarray([scale], jnp.float32))


def _sc_conv_pair(hA, srcA, dstA, wA, hB, srcB, dstB, wB):
    n, f = hA.shape
    n_chunks = srcA.shape[0] // NS
    rows_per_tile = n // NS
    mesh = plsc.VectorSubcoreMesh(core_axis_name="c", subcore_axis_name="s")

    @functools.partial(
        pl.kernel,
        out_type=(
            jax.ShapeDtypeStruct((n, f), jnp.float32),
            jax.ShapeDtypeStruct((n, f), jnp.float32),
        ),
        mesh=mesh,
        scratch_types=[
            pltpu.VMEM((n_chunks, CHUNK), jnp.int32),
            pltpu.VMEM((n_chunks, CHUNK), jnp.int32),
            pltpu.VMEM((n_chunks, CHUNK), jnp.float32),
            pltpu.VMEM((CHUNK, f), jnp.float32),
            pltpu.VMEM_SHARED((n, f), jnp.float32),
            pltpu.SemaphoreType.DMA,
        ],
    )
    def k(hA_ref, srcA_ref, dstA_ref, wA_ref, hB_ref, srcB_ref, dstB_ref, wB_ref,
          outA_ref, outB_ref, src_v, dst_v, w_v, rows_v, acc, sem):
        c = lax.axis_index("c")
        s = lax.axis_index("s")
        zero16 = jnp.zeros((LANES,), jnp.float32)

        def run(h_ref, src_ref, dst_ref, w_ref, out_ref):
            base = s * n_chunks
            pltpu.sync_copy(src_ref.at[pl.ds(base, n_chunks)], src_v)
            pltpu.sync_copy(dst_ref.at[pl.ds(base, n_chunks)], dst_v)
            pltpu.sync_copy(w_ref.at[pl.ds(base, n_chunks)], w_v)

            def zrow(i, carry):
                for g in range(f // LANES):
                    rows_v[i, pl.ds(g * LANES, LANES)] = zero16
                return carry
            lax.fori_loop(0, CHUNK, zrow, 0)
            row0 = s * rows_per_tile
            nfull = rows_per_tile // CHUNK
            for blk in range(nfull):
                pltpu.sync_copy(rows_v, acc.at[pl.ds(row0 + blk * CHUNK, CHUNK)])
            rem = rows_per_tile - nfull * CHUNK
            if rem:
                pltpu.sync_copy(rows_v.at[pl.ds(0, rem)],
                                acc.at[pl.ds(row0 + nfull * CHUNK, rem)])
            plsc.subcore_barrier()

            def chunk_body(j, carry):
                pltpu.async_copy(h_ref.at[src_v.at[j]], rows_v, sem).wait()

                def edge_body(e, c2):
                    jj = jnp.full((LANES,), j, jnp.int32)
                    ee = jnp.full((LANES,), e, jnp.int32)
                    wv = plsc.load_gather(w_v, [jj, ee])
                    for g in range(f // LANES):
                        sl = rows_v[e, pl.ds(g * LANES, LANES)]
                        rows_v[e, pl.ds(g * LANES, LANES)] = sl * wv
                    return c2
                lax.fori_loop(0, CHUNK, edge_body, 0)
                pltpu.sync_copy(rows_v, acc.at[dst_v.at[j]], add=True)
                return carry
            lax.fori_loop(0, n_chunks, chunk_body, 0)
            plsc.subcore_barrier()
            pltpu.sync_copy(acc.at[pl.ds(row0, rows_per_tile)],
                            out_ref.at[pl.ds(row0, rows_per_tile)])

        @pl.when(c == 0)
        def _():
            run(hA_ref, srcA_ref, dstA_ref, wA_ref, outA_ref)

        @pl.when(c == 1)
        def _():
            run(hB_ref, srcB_ref, dstB_ref, wB_ref, outB_ref)

    return k(hA, srcA, dstA, wA, hB, srcB, dstB, wB)


def _prep_edges(ei, ew):
    e = ei.shape[1]
    n_chunks = -(-e // (NS * CHUNK))
    epad = NS * n_chunks * CHUNK
    pad = epad - e
    src = jnp.concatenate([ei[0], jnp.zeros((pad,), jnp.int32)]).reshape(NS * n_chunks, CHUNK)
    dst = jnp.concatenate([ei[1], jnp.zeros((pad,), jnp.int32)]).reshape(NS * n_chunks, CHUNK)
    w = jnp.concatenate([ew, jnp.zeros((pad,), jnp.float32)]).reshape(NS * n_chunks, CHUNK)
    return src, dst, w


def kernel(features, edge_index, edge_weight, edge_index2, edge_weight2,
           W_ln1, b_ln1, W_c1a, b_c1a, W_c1b, b_c1b, bn1_g, bn1_b,
           W_ln2, b_ln2, W_c2a, b_c2a, W_c2b, b_c2b, bn2_g, bn2_b,
           W_conv, b_conv):
    srcA, dstA, wA = _prep_edges(edge_index, edge_weight)
    srcB, dstB, wB = _prep_edges(edge_index2, edge_weight2)

    x0, ha, hb = _t1(features, W_ln1, b_ln1, W_c1a, W_c1b)
    pA, pB = _sc_conv_pair(ha, srcA, dstA, wA, hb, srcB, dstB, wB)
    y0, h2a, h2b = _t2(x0, pA, pB, b_c1a + b_c1b, 2.0, bn1_g, bn1_b,
                       W_ln2, b_ln2, W_c2a, W_c2b)
    qA, qB = _sc_conv_pair(h2a, srcA, dstA, wA, h2b, srcB, dstB, wB)
    out = _t3(y0, qA, qB, b_c2a + b_c2b, 0.5, bn2_g, bn2_b,
              W_conv.T, b_conv)
    return out


# speedup vs baseline: 4.9622x; 4.9622x over previous
import functools

import jax
import jax.numpy as jnp
from jax import lax
from jax.experimental import pallas as pl
from jax.experimental.pallas import tpu as pltpu
from jax.experimental.pallas import tpu_sc as plsc

NS = 16
LANES = 16
CHUNK = 128


def _t1_body(x_ref, w0_ref, b0_ref, wa_ref, wb_ref, o0_ref, oa_ref, ob_ref):
    x = x_ref[...]
    o0_ref[...] = jnp.dot(x, w0_ref[...], preferred_element_type=jnp.float32) + b0_ref[...]
    oa_ref[...] = jnp.dot(x, wa_ref[...], preferred_element_type=jnp.float32)
    ob_ref[...] = jnp.dot(x, wb_ref[...], preferred_element_type=jnp.float32)


def _t1(x, W0, b0, Wa, Wb):
    n, d = x.shape
    h = W0.shape[1]
    blk = 2000
    grid = n // blk
    return pl.pallas_call(
        _t1_body,
        grid=(grid,),
        in_specs=[
            pl.BlockSpec((blk, d), lambda i: (i, 0)),
            pl.BlockSpec((d, h), lambda i: (0, 0)),
            pl.BlockSpec((1, h), lambda i: (0, 0)),
            pl.BlockSpec((d, h), lambda i: (0, 0)),
            pl.BlockSpec((d, h), lambda i: (0, 0)),
        ],
        out_specs=[
            pl.BlockSpec((blk, h), lambda i: (i, 0)),
            pl.BlockSpec((blk, h), lambda i: (i, 0)),
            pl.BlockSpec((blk, h), lambda i: (i, 0)),
        ],
        out_shape=[
            jax.ShapeDtypeStruct((n, h), jnp.float32),
            jax.ShapeDtypeStruct((n, h), jnp.float32),
            jax.ShapeDtypeStruct((n, h), jnp.float32),
        ],
    )(x, W0, b0.reshape(1, h), Wa, Wb)


def _bn_mm3_body(x0_ref, pa_ref, pb_ref, bsum_ref, g_ref, bb_ref,
                 w0_ref, b0_ref, wa_ref, wb_ref, scale_ref,
                 o0_ref, oa_ref, ob_ref):
    left = x0_ref[...]
    nrows = left.shape[0]
    right = scale_ref[0] * (pa_ref[...][:nrows] + pb_ref[...][:nrows] + bsum_ref[...])
    u = jnp.concatenate([left, right], axis=1)
    mu = jnp.sum(u, axis=0, keepdims=True) / nrows
    xc = u - mu
    var = jnp.sum(xc * xc, axis=0, keepdims=True) / nrows
    z = xc * lax.rsqrt(var + 1e-5) * g_ref[...] + bb_ref[...]
    o0_ref[...] = jnp.dot(z, w0_ref[...], preferred_element_type=jnp.float32) + b0_ref[...]
    oa_ref[...] = jnp.dot(z, wa_ref[...], preferred_element_type=jnp.float32)
    ob_ref[...] = jnp.dot(z, wb_ref[...], preferred_element_type=jnp.float32)


def _t2(x0, pa, pb, bsum, scale, g, bb, W0, b0, Wa, Wb):
    n, h = x0.shape
    c = W0.shape[1]
    return pl.pallas_call(
        _bn_mm3_body,
        out_shape=[
            jax.ShapeDtypeStruct((n, c), jnp.float32),
            jax.ShapeDtypeStruct((n, c), jnp.float32),
            jax.ShapeDtypeStruct((n, c), jnp.float32),
        ],
    )(x0, pa, pb, bsum.reshape(1, h), g.reshape(1, 2 * h), bb.reshape(1, 2 * h),
      W0, b0.reshape(1, c), Wa, Wb, jnp.asarray([scale], jnp.float32))


def _bn_final_body(x0_ref, pa_ref, pb_ref, bsum_ref, g_ref, bb_ref,
                   wt_ref, bout_ref, scale_ref, o_ref):
    left = x0_ref[...]
    nrows = left.shape[0]
    right = scale_ref[0] * (pa_ref[...][:nrows] + pb_ref[...][:nrows] + bsum_ref[...])
    u = jnp.concatenate([left, right], axis=1)
    mu = jnp.sum(u, axis=0, keepdims=True) / nrows
    xc = u - mu
    var = jnp.sum(xc * xc, axis=0, keepdims=True) / nrows
    z = xc * lax.rsqrt(var + 1e-5) * g_ref[...] + bb_ref[...]
    o_ref[...] = jnp.dot(z, wt_ref[...], preferred_element_type=jnp.float32) + bout_ref[...]


def _t3(x0, pa, pb, bsum, scale, g, bb, Wt, bout):
    n, c = x0.shape
    cout = Wt.shape[1]
    return pl.pallas_call(
        _bn_final_body,
        out_shape=jax.ShapeDtypeStruct((n, cout), jnp.float32),
    )(x0, pa, pb, bsum.reshape(1, c), g.reshape(1, 2 * c), bb.reshape(1, 2 * c),
      Wt, bout.reshape(1, cout), jnp.asarray([scale], jnp.float32))


def _sc_conv_pair(hA, srcA, dstA, wA, hB, srcB, dstB, wB):
    n, f = hA.shape
    n_chunks = srcA.shape[0] // NS
    n_acc = -(-n // (NS * 8)) * (NS * 8)
    rows_per_tile = n_acc // NS
    mesh = plsc.VectorSubcoreMesh(core_axis_name="c", subcore_axis_name="s")

    @functools.partial(
        pl.kernel,
        out_type=(
            jax.ShapeDtypeStruct((n_acc, f), jnp.float32),
            jax.ShapeDtypeStruct((n_acc, f), jnp.float32),
        ),
        mesh=mesh,
        compiler_params=pltpu.CompilerParams(use_tc_tiling_on_sc=False),
        scratch_types=[
            pltpu.VMEM((n_chunks, CHUNK), jnp.int32),
            pltpu.VMEM((n_chunks, CHUNK), jnp.int32),
            pltpu.VMEM((n_chunks * CHUNK,), jnp.float32),
            pltpu.VMEM((CHUNK, f), jnp.float32),
            pltpu.VMEM_SHARED((n_acc, f), jnp.float32),
            pltpu.SemaphoreType.DMA,
        ],
    )
    def k(hA_ref, srcA_ref, dstA_ref, wA_ref, hB_ref, srcB_ref, dstB_ref, wB_ref,
          outA_ref, outB_ref, src_v, dst_v, w_v, rows_v, acc, sem):
        c = lax.axis_index("c")
        s = lax.axis_index("s")
        zero16 = jnp.zeros((LANES,), jnp.float32)

        def run(h_ref, src_ref, dst_ref, w_ref, out_ref):
            base = s * n_chunks
            pltpu.sync_copy(src_ref.at[pl.ds(base, n_chunks)], src_v)
            pltpu.sync_copy(dst_ref.at[pl.ds(base, n_chunks)], dst_v)
            pltpu.sync_copy(w_ref.at[pl.ds(base * CHUNK, n_chunks * CHUNK)], w_v)

            def zrow(i, carry):
                for g in range(f // LANES):
                    rows_v[i, pl.ds(g * LANES, LANES)] = zero16
                return carry
            lax.fori_loop(0, CHUNK, zrow, 0)
            row0 = s * rows_per_tile
            nfull = rows_per_tile // CHUNK
            for blk in range(nfull):
                pltpu.sync_copy(rows_v, acc.at[pl.ds(row0 + blk * CHUNK, CHUNK)])
            rem = rows_per_tile - nfull * CHUNK
            if rem:
                pltpu.sync_copy(rows_v.at[pl.ds(0, rem)],
                                acc.at[pl.ds(row0 + nfull * CHUNK, rem)])
            plsc.subcore_barrier()

            def chunk_body(j, carry):
                pltpu.async_copy(h_ref.at[src_v.at[j]], rows_v, sem).wait()

                def group_body(ge, c2):
                    wg = w_v[pl.ds(j * CHUNK + ge * LANES, LANES)]
                    for i in range(LANES):
                        wv = jnp.full((LANES,), wg[i])
                        e = ge * LANES + i
                        for g in range(f // LANES):
                            sl = rows_v[e, pl.ds(g * LANES, LANES)]
                            rows_v[e, pl.ds(g * LANES, LANES)] = sl * wv
                    return c2
                lax.fori_loop(0, CHUNK // LANES, group_body, 0)
                pltpu.sync_copy(rows_v, acc.at[dst_v.at[j]], add=True)
                return carry
            lax.fori_loop(0, n_chunks, chunk_body, 0)
            plsc.subcore_barrier()
            pltpu.sync_copy(acc.at[pl.ds(row0, rows_per_tile)],
                            out_ref.at[pl.ds(row0, rows_per_tile)])

        @pl.when(c == 0)
        def _():
            run(hA_ref, srcA_ref, dstA_ref, wA_ref, outA_ref)

        @pl.when(c == 1)
        def _():
            run(hB_ref, srcB_ref, dstB_ref, wB_ref, outB_ref)

    return k(hA, srcA, dstA, wA, hB, srcB, dstB, wB)


def _prep_edges(ei, ew):
    e = ei.shape[1]
    n_chunks = -(-e // (NS * CHUNK))
    n_chunks = -(-n_chunks // 8) * 8
    epad = NS * n_chunks * CHUNK
    pad = epad - e
    src = jnp.concatenate([ei[0], jnp.zeros((pad,), jnp.int32)]).reshape(NS * n_chunks, CHUNK)
    dst = jnp.concatenate([ei[1], jnp.zeros((pad,), jnp.int32)]).reshape(NS * n_chunks, CHUNK)
    w = jnp.concatenate([ew, jnp.zeros((pad,), jnp.float32)])
    return src, dst, w


def kernel(features, edge_index, edge_weight, edge_index2, edge_weight2,
           W_ln1, b_ln1, W_c1a, b_c1a, W_c1b, b_c1b, bn1_g, bn1_b,
           W_ln2, b_ln2, W_c2a, b_c2a, W_c2b, b_c2b, bn2_g, bn2_b,
           W_conv, b_conv):
    srcA, dstA, wA = _prep_edges(edge_index, edge_weight)
    srcB, dstB, wB = _prep_edges(edge_index2, edge_weight2)

    x0, ha, hb = _t1(features, W_ln1, b_ln1, W_c1a, W_c1b)
    pA, pB = _sc_conv_pair(ha, srcA, dstA, wA, hb, srcB, dstB, wB)
    y0, h2a, h2b = _t2(x0, pA, pB, b_c1a + b_c1b, 2.0, bn1_g, bn1_b,
                       W_ln2, b_ln2, W_c2a, W_c2b)
    qA, qB = _sc_conv_pair(h2a, srcA, dstA, wA, h2b, srcB, dstB, wB)
    out = _t3(y0, qA, qB, b_c2a + b_c2b, 0.5, bn2_g, bn2_b,
              W_conv.T, b_conv)
    return out
